# Initial kernel scaffold; baseline (speedup 1.0000x reference)
import jax
import jax.numpy as jnp
from jax.experimental import pallas as pl


def kernel(input_vertex_features, input_vertex_coordinates, keypoint_indices, edges, ao_params, edge_params, update_params):
    raise NotImplementedError("write your pallas kernel here")


# speedup vs baseline: 1.6019x; 1.6019x over previous
import functools

import jax
import jax.numpy as jnp
from jax import lax
from jax.experimental import pallas as pl
from jax.experimental.pallas import tpu as pltpu
from jax.experimental.pallas import tpu_sc as plsc

N = 10000
E = 320000
D = 128
EPS = 1e-5

NC = 2
NS = 16
NW = NC * NS
E_PER = E // NW
CH1 = 80
NCH1 = E_PER // CH1
NLOC = 320
CH2 = 2000
NCH2 = E // CH2
BE = 2000
NBE = E // BE


def _bn_train(x, g, b):
    m = jnp.mean(x, axis=0, keepdims=True)
    v = jnp.mean((x - m) * (x - m), axis=0, keepdims=True)
    return (x - m) / jnp.sqrt(v + EPS) * g + b


def _tc_pre_body(f_ref, c_ref, wa1t, ba1, ga1, bta1, wa2t, ba2, ga2, bta2,
                 w1at, w1bt, b1, p_ref, q_ref):
    f = f_ref[...]
    c = c_ref[...]
    x = jnp.maximum(jnp.dot(f, wa1t[...], preferred_element_type=jnp.float32)
                    + ba1[...], 0.0)
    x = _bn_train(x, ga1[...], bta1[...])
    x = jnp.maximum(jnp.dot(x, wa2t[...], preferred_element_type=jnp.float32)
                    + ba2[...], 0.0)
    off = _bn_train(x, ga2[...], bta2[...])
    c2 = c + off
    p_ref[...] = (jnp.dot(f, w1at[...], preferred_element_type=jnp.float32)
                  + jnp.dot(c, w1bt[...], preferred_element_type=jnp.float32)
                  + b1[...])
    q_ref[...] = jnp.dot(c2, w1bt[...], preferred_element_type=jnp.float32)


def _tc_pre(f, c, wa1t, ba1, ga1, bta1, wa2t, ba2, ga2, bta2, w1at, w1bt, b1):
    return pl.pallas_call(
        _tc_pre_body,
        out_shape=[jax.ShapeDtypeStruct((N, D), jnp.float32),
                   jax.ShapeDtypeStruct((N, D), jnp.float32)],
    )(f, c, wa1t, ba1, ga1, bta1, wa2t, ba2, ga2, bta2, w1at, w1bt, b1)


def _sc1_body(src_hbm, dst_hbm, p_hbm, q_hbm, h_hbm, sh_hbm, sq_hbm,
              srcv, dstv, pv, qv, hv, shv, sqv, sem_a, sem_b):
    wid = lax.axis_index("s") * NC + lax.axis_index("c")
    base0 = wid * E_PER
    zero = jnp.zeros((16,), jnp.float32)
    init = (tuple(zero for _ in range(8)), tuple(zero for _ in range(8)))

    def chunk(ci, carry):
        base = base0 + ci * CH1
        cp1 = pltpu.async_copy(src_hbm.at[pl.ds(base, CH1)], srcv, sem_a)
        cp2 = pltpu.async_copy(dst_hbm.at[pl.ds(base, CH1)], dstv, sem_b)
        cp1.wait()
        cp2.wait()
        g1 = pltpu.async_copy(p_hbm.at[srcv], pv, sem_a)
        g2 = pltpu.async_copy(q_hbm.at[dstv], qv, sem_b)
        g1.wait()
        g2.wait()

        def row(i, cr):
            sh, sq = cr
            nsh = []
            nsq = []
            for k in range(8):
                sl = pl.ds(16 * k, 16)
                h = jnp.maximum(pv[i, sl] - qv[i, sl], 0.0)
                hv[i, sl] = h
                nsh.append(sh[k] + h)
                nsq.append(sq[k] + h * h)
            return (tuple(nsh), tuple(nsq))

        carry = lax.fori_loop(0, CH1, row, carry)
        pltpu.sync_copy(hv, h_hbm.at[pl.ds(base, CH1)])
        return carry

    sh, sq = lax.fori_loop(0, NCH1, chunk, init)
    for k in range(8):
        sl = pl.ds(16 * k, 16)
        shv[0, sl] = sh[k]
        sqv[0, sl] = sq[k]
    pltpu.sync_copy(shv, sh_hbm.at[pl.ds(wid, 1)])
    pltpu.sync_copy(sqv, sq_hbm.at[pl.ds(wid, 1)])


def _sc_pass1(src, dst, p, q):
    mesh = plsc.VectorSubcoreMesh(core_axis_name="c", subcore_axis_name="s")
    return pl.kernel(
        _sc1_body,
        out_type=[jax.ShapeDtypeStruct((E, D), jnp.float32),
                  jax.ShapeDtypeStruct((NW, D), jnp.float32),
                  jax.ShapeDtypeStruct((NW, D), jnp.float32)],
        mesh=mesh,
        compiler_params=pltpu.CompilerParams(needs_layout_passes=False),
        scratch_types=[pltpu.VMEM((CH1,), jnp.int32),
                       pltpu.VMEM((CH1,), jnp.int32),
                       pltpu.VMEM((CH1, D), jnp.float32),
                       pltpu.VMEM((CH1, D), jnp.float32),
                       pltpu.VMEM((CH1, D), jnp.float32),
                       pltpu.VMEM((1, D), jnp.float32),
                       pltpu.VMEM((1, D), jnp.float32),
                       pltpu.SemaphoreType.DMA,
                       pltpu.SemaphoreType.DMA],
    )(src, dst, p, q)


def _tc_z_body(h_ref, sh_ref, sq_ref, g1, bt1, w2t, b2, z_ref, sz_ref, szz_ref):
    i = pl.program_id(0)
    m1 = jnp.sum(sh_ref[...], axis=0, keepdims=True) * (1.0 / E)
    v1 = jnp.sum(sq_ref[...], axis=0, keepdims=True) * (1.0 / E) - m1 * m1
    inv1 = g1[...] / jnp.sqrt(v1 + EPS)
    hn = (h_ref[...] - m1) * inv1 + bt1[...]
    z = jnp.maximum(jnp.dot(hn, w2t[...], preferred_element_type=jnp.float32)
                    + b2[...], 0.0)
    z_ref[...] = z
    bs = jnp.sum(z, axis=0, keepdims=True)
    bss = jnp.sum(z * z, axis=0, keepdims=True)

    @pl.when(i == 0)
    def _():
        sz_ref[...] = bs
        szz_ref[...] = bss

    @pl.when(i > 0)
    def _():
        sz_ref[...] += bs
        szz_ref[...] += bss


def _tc_z(h, sh, sq, g1, bt1, w2t, b2):
    return pl.pallas_call(
        _tc_z_body,
        grid=(NBE,),
        in_specs=[pl.BlockSpec((BE, D), lambda i: (i, 0)),
                  pl.BlockSpec((NW, D), lambda i: (0, 0)),
                  pl.BlockSpec((NW, D), lambda i: (0, 0)),
                  pl.BlockSpec((1, D), lambda i: (0, 0)),
                  pl.BlockSpec((1, D), lambda i: (0, 0)),
                  pl.BlockSpec((D, D), lambda i: (0, 0)),
                  pl.BlockSpec((1, D), lambda i: (0, 0))],
        out_specs=[pl.BlockSpec((BE, D), lambda i: (i, 0)),
                   pl.BlockSpec((1, D), lambda i: (0, 0)),
                   pl.BlockSpec((1, D), lambda i: (0, 0))],
        out_shape=[jax.ShapeDtypeStruct((E, D), jnp.float32),
                   jax.ShapeDtypeStruct((1, D), jnp.float32),
                   jax.ShapeDtypeStruct((1, D), jnp.float32)],
    )(h, sh, sq, g1, bt1, w2t, b2)


def _sc2_body(dst_hbm, z_hbm, r_hbm, dstv, sele, seld, zbuf, acc, sem_g):
    wid = lax.axis_index("s") * NC + lax.axis_index("c")
    lo = wid * NLOC
    zero = jnp.zeros((16,), jnp.float32)

    def zrow(i, _):
        for k in range(8):
            acc[i, pl.ds(16 * k, 16)] = zero
        return 0

    lax.fori_loop(0, NLOC + 1, zrow, 0)
    iota16 = lax.iota(jnp.int32, 16)

    def chunk(ci, _):
        cbase = ci * CH2
        pltpu.sync_copy(dst_hbm.at[pl.ds(cbase, CH2)], dstv)

        def scan(g, cur):
            v = dstv[pl.ds(g * 16, 16)]
            dloc = v - lo
            m = (dloc >= 0) & (dloc < NLOC)
            eid = (cbase + g * 16) + iota16
            mi = jnp.where(m, 1, 0)
            pos = cur + plsc.cumsum(mi) - mi
            plsc.store_scatter(sele, [pos], eid, mask=m)
            plsc.store_scatter(seld, [pos], dloc, mask=m)
            return cur + plsc.all_reduce_population_count(m)[0]

        cur = lax.fori_loop(0, CH2 // 16, scan, 0)
        sele[pl.ds(cur, 16)] = jnp.zeros((16,), jnp.int32)
        seld[pl.ds(cur, 16)] = jnp.full((16,), NLOC, jnp.int32)
        ng = (cur + 15) // 16

        def proc(j, _):
            pltpu.async_copy(z_hbm.at[sele.at[pl.ds(j * 16, 16)]], zbuf,
                             sem_g).wait()
            dv = seld[pl.ds(j * 16, 16)]
            for i in range(16):
                d = dv[i]
                for k in range(8):
                    sl = pl.ds(16 * k, 16)
                    acc[d, sl] = jnp.maximum(acc[d, sl], zbuf[i, sl])
            return 0

        lax.fori_loop(0, ng, proc, 0)
        return 0

    lax.fori_loop(0, NCH2, chunk, 0)
    pltpu.sync_copy(acc.at[pl.ds(0, NLOC)], r_hbm.at[pl.ds(lo, NLOC)])


def _sc_pass2(dst, z):
    mesh = plsc.VectorSubcoreMesh(core_axis_name="c", subcore_axis_name="s")
    return pl.kernel(
        _sc2_body,
        out_type=jax.ShapeDtypeStruct((NW * NLOC, D), jnp.float32),
        mesh=mesh,
        compiler_params=pltpu.CompilerParams(needs_layout_passes=False),
        scratch_types=[pltpu.VMEM((CH2,), jnp.int32),
                       pltpu.VMEM((CH2 + 16,), jnp.int32),
                       pltpu.VMEM((CH2 + 16,), jnp.int32),
                       pltpu.VMEM((16, D), jnp.float32),
                       pltpu.VMEM((NLOC + 1, D), jnp.float32),
                       pltpu.SemaphoreType.DMA],
    )(dst, z)


def _tc_post_body(r_ref, f_ref, sz_ref, szz_ref, g2, bt2,
                  wu1t, bu1, gu1, btu1, wu2t, bu2, gu2, btu2, out_ref):
    m2 = sz_ref[...] * (1.0 / E)
    v2 = szz_ref[...] * (1.0 / E) - m2 * m2
    inv2 = g2[...] / jnp.sqrt(v2 + EPS)
    r = r_ref[...]
    agg = jnp.maximum((r - m2) * inv2 + bt2[...], 0.0)
    x = jnp.maximum(jnp.dot(agg, wu1t[...], preferred_element_type=jnp.float32)
                    + bu1[...], 0.0)
    x = _bn_train(x, gu1[...], btu1[...])
    x = jnp.maximum(jnp.dot(x, wu2t[...], preferred_element_type=jnp.float32)
                    + bu2[...], 0.0)
    x = _bn_train(x, gu2[...], btu2[...])
    out_ref[...] = x + f_ref[...]


def _tc_post(r, f, sz, szz, g2, bt2, wu1t, bu1, gu1, btu1, wu2t, bu2, gu2,
             btu2):
    return pl.pallas_call(
        _tc_post_body,
        out_shape=jax.ShapeDtypeStruct((N, D), jnp.float32),
    )(r, f, sz, szz, g2, bt2, wu1t, bu1, gu1, btu1, wu2t, bu2, gu2, btu2)


def kernel(input_vertex_features, input_vertex_coordinates, keypoint_indices,
           edges, ao_params, edge_params, update_params):
    f = input_vertex_features
    c = input_vertex_coordinates
    del keypoint_indices

    (wa1, ba1, ga1, bta1), (wa2, ba2, ga2, bta2) = ao_params
    (w1, b1, g1, bt1), (w2, b2, g2, bt2) = edge_params
    (wu1, bu1, gu1, btu1), (wu2, bu2, gu2, btu2) = update_params

    row = lambda v: v.reshape(1, -1)
    src = edges[:, 0]
    dst = edges[:, 1]

    p, q = _tc_pre(f, c, wa1.T, row(ba1), row(ga1), row(bta1),
                   wa2.T, row(ba2), row(ga2), row(bta2),
                   w1[:, :D].T, w1[:, D:].T, row(b1))

    h, sh, sq = _sc_pass1(src, dst, p, q)

    z, sz, szz = _tc_z(h, sh, sq, row(g1), row(bt1), w2.T, row(b2))

    r_full = _sc_pass2(dst, z)

    out = _tc_post(r_full[:N], f, sz, szz, row(g2), row(bt2),
                   wu1.T, row(bu1), row(gu1), row(btu1),
                   wu2.T, row(bu2), row(gu2), row(btu2))
    return out
